# baseline (speedup 1.0000x reference)
import functools

import jax
import jax.numpy as jnp
from jax.experimental import pallas as pl
from jax.experimental.pallas import tpu as pltpu

K = 50
NUM_BINS = 5
CUTOFF = 10.0
BR = 128
R = 7


def _knn_kernel(m, npad, pa_ref, prt_ref, idx_ref, attr_ref):
    nc = npad // 128
    b = pl.program_id(0)
    pa = pa_ref[...]
    prt = prt_ref[...]
    sq_all = jnp.sum(pa * pa, axis=1, keepdims=True)
    sq_r = jnp.sum(prt * prt, axis=0, keepdims=True)
    g = jax.lax.dot_general(pa, prt, (((1,), (0,)), ((), ())),
                            preferred_element_type=jnp.float32)
    d2 = sq_all + sq_r - 2.0 * g
    d2_3 = d2.reshape(nc, 128, 128)
    ci3 = (jax.lax.broadcasted_iota(jnp.int32, (nc, 128, 128), 0) * 128
           + jax.lax.broadcasted_iota(jnp.int32, (nc, 128, 128), 1))
    gi = b * 128 + jax.lax.broadcasted_iota(jnp.int32, (nc, 128, 128), 2)
    d2_3 = jnp.where(ci3 == gi, d2_3 + 1e10, d2_3)
    d2_3 = jnp.where(ci3 >= m, jnp.float32(1e30), d2_3)
    cf3 = ci3.astype(jnp.float32)
    npf = jnp.float32(npad)

    inf = jnp.float32(1e30)
    vals = d2_3
    pools_v = []
    pools_c = []
    for _ in range(R):
        cm = jnp.min(vals, axis=1)
        carg = jnp.min(jnp.where(vals == cm[:, None, :], cf3, npf),
                       axis=1)
        pools_v.append(cm)
        pools_c.append(carg)
        vals = jnp.where(cf3 == carg[:, None, :], inf, vals)
    poolv = jnp.concatenate(pools_v, axis=0)
    poolc = jnp.concatenate(pools_c, axis=0)
    m_star = jnp.min(jnp.min(vals, axis=1), axis=0, keepdims=True)

    top_i = []
    top_d = []
    pv = poolv
    for _ in range(K):
        mn = jnp.min(pv, axis=0, keepdims=True)
        arg = jnp.min(jnp.where(pv == mn, poolc, npf), axis=0,
                      keepdims=True)
        top_i.append(arg)
        top_d.append(mn)
        pv = jnp.where(poolc == arg, inf, pv)
    topi = jnp.concatenate(top_i, axis=0)
    topd = jnp.concatenate(top_d, axis=0)
    valid = jnp.all(topd[K - 1:K, :] < m_star)

    ki = jax.lax.broadcasted_iota(jnp.int32, (K, 128), 0)

    def pool_path():
        return topi, topd

    def exact_path():
        def body(k, carry):
            v, ti, td = carry
            mn = jnp.min(jnp.min(v, axis=1), axis=0, keepdims=True)
            arg = jnp.min(jnp.min(
                jnp.where(v == mn[None, :, :], cf3, npf), axis=1),
                axis=0, keepdims=True)
            ti = jnp.where(ki == k, arg, ti)
            td = jnp.where(ki == k, mn, td)
            v = jnp.where(cf3 == arg[None, :, :], inf, v)
            return v, ti, td
        zi = jnp.zeros((K, 128), jnp.float32)
        zd = jnp.zeros((K, 128), jnp.float32)
        _, ti, td = jax.lax.fori_loop(0, K, body, (d2_3, zi, zd))
        return ti, td

    topi, topd = jax.lax.cond(valid, pool_path, exact_path)

    idx_ref[...] = topi.astype(jnp.int32)
    dist = jnp.sqrt(jnp.maximum(topd, 0.0) + 1e-12)
    centers = jax.lax.broadcasted_iota(
        jnp.int32, (1, NUM_BINS, 1), 1).astype(jnp.float32) * 2.5
    two_s2 = jnp.float32(12.5)
    attr_ref[...] = jnp.exp(-((dist[:, None, :] - centers) ** 2) / two_s2)


def kernel(pos, keep_idx):
    p = pos[keep_idx]
    M = p.shape[0]
    npad = ((M + 127) // 128) * 128
    pa = jnp.pad(p, ((0, npad - M), (0, 0)))
    prt = pa.T
    grid = (npad // 128,)
    nbr_t, attr_t = pl.pallas_call(
        functools.partial(_knn_kernel, M, npad),
        grid=grid,
        in_specs=[
            pl.BlockSpec((npad, 3), lambda b: (0, 0)),
            pl.BlockSpec((3, BR), lambda b: (0, b)),
        ],
        out_specs=[
            pl.BlockSpec((K, BR), lambda b: (0, b)),
            pl.BlockSpec((K, NUM_BINS, BR), lambda b: (0, 0, b)),
        ],
        out_shape=[
            jax.ShapeDtypeStruct((K, npad), jnp.int32),
            jax.ShapeDtypeStruct((K, NUM_BINS, npad), jnp.float32),
        ],
        compiler_params=pltpu.CompilerParams(
            dimension_semantics=("parallel",)),
    )(pa, prt)

    nbr = nbr_t[:, :M].T
    src = nbr.reshape(-1)
    dst = jnp.repeat(jnp.arange(M, dtype=jnp.int32), K)
    edge_index = jnp.stack([jnp.concatenate([src, dst]),
                            jnp.concatenate([dst, src])])
    A = attr_t[:, :, :M].transpose(2, 0, 1).reshape(-1, NUM_BINS)
    edge_attr = jnp.concatenate([A, A], axis=0)
    return p, edge_index, edge_attr


# speedup vs baseline: 1.7435x; 1.7435x over previous
import functools

import jax
import jax.numpy as jnp
from jax.experimental import pallas as pl
from jax.experimental.pallas import tpu as pltpu

K = 50
NUM_BINS = 5
CUTOFF = 10.0
BR = 128
R = 8


def _knn_kernel(m, npad, pa_ref, prt_ref, idx_ref, attr_ref):
    nc = npad // 128
    b = pl.program_id(0)
    pa = pa_ref[...]
    prt = prt_ref[...]
    sq_all = jnp.sum(pa * pa, axis=1, keepdims=True)
    sq_r = jnp.sum(prt * prt, axis=0, keepdims=True)
    g = jax.lax.dot_general(pa, prt, (((1,), (0,)), ((), ())),
                            preferred_element_type=jnp.float32)
    d2 = sq_all + sq_r - 2.0 * g
    if npad > m:
        d2 = jnp.concatenate(
            [d2[:m], jnp.full((npad - m, 128), 1e30, jnp.float32)], axis=0)
    d2_3 = d2.reshape(nc, 128, 128)
    cf3 = (jax.lax.broadcasted_iota(jnp.int32, (nc, 128, 128), 0) * 128
           + jax.lax.broadcasted_iota(jnp.int32, (nc, 128, 128), 1)
           ).astype(jnp.float32)
    rowf = (jnp.float32(b * 128)
            + jax.lax.broadcasted_iota(jnp.int32, (1, 1, 128), 2
                                       ).astype(jnp.float32))
    d2_3 = jnp.where(cf3 == rowf, d2_3 + 1e10, d2_3)
    npf = jnp.float32(npad)

    inf = jnp.float32(1e30)
    vals = d2_3
    pools_v = []
    pools_c = []
    for _ in range(R):
        cm = jnp.min(vals, axis=1)
        carg = jnp.min(jnp.where(vals == cm[:, None, :], cf3, npf),
                       axis=1)
        pools_v.append(cm)
        pools_c.append(carg)
        vals = jnp.where(cf3 == carg[:, None, :], inf, vals)
    poolv = jnp.concatenate(pools_v, axis=0)
    poolc = jnp.concatenate(pools_c, axis=0)
    m_star = jnp.min(jnp.min(vals, axis=1), axis=0, keepdims=True)

    top_i = []
    top_d = []
    pv = poolv
    for _ in range(K):
        mn = jnp.min(pv, axis=0, keepdims=True)
        arg = jnp.min(jnp.where(pv == mn, poolc, npf), axis=0,
                      keepdims=True)
        top_i.append(arg)
        top_d.append(mn)
        pv = jnp.where(poolc == arg, inf, pv)
    topi = jnp.concatenate(top_i, axis=0)
    topd = jnp.concatenate(top_d, axis=0)
    valid = jnp.all(topd[K - 1:K, :] < m_star)

    ki = jax.lax.broadcasted_iota(jnp.int32, (K, 128), 0)

    def pool_path():
        return topi, topd

    def exact_path():
        def body(k, carry):
            v, ti, td = carry
            mn = jnp.min(jnp.min(v, axis=1), axis=0, keepdims=True)
            arg = jnp.min(jnp.min(
                jnp.where(v == mn[None, :, :], cf3, npf), axis=1),
                axis=0, keepdims=True)
            ti = jnp.where(ki == k, arg, ti)
            td = jnp.where(ki == k, mn, td)
            v = jnp.where(cf3 == arg[None, :, :], inf, v)
            return v, ti, td
        zi = jnp.zeros((K, 128), jnp.float32)
        zd = jnp.zeros((K, 128), jnp.float32)
        _, ti, td = jax.lax.fori_loop(0, K, body, (d2_3, zi, zd))
        return ti, td

    topi, topd = jax.lax.cond(valid, pool_path, exact_path)

    idx_ref[...] = topi.astype(jnp.int32)
    dist = jnp.sqrt(jnp.maximum(topd, 0.0) + 1e-12)
    centers = jax.lax.broadcasted_iota(
        jnp.int32, (1, NUM_BINS, 1), 1).astype(jnp.float32) * 2.5
    two_s2 = jnp.float32(12.5)
    attr_ref[...] = jnp.exp(-((dist[:, None, :] - centers) ** 2) / two_s2)


def kernel(pos, keep_idx):
    p = pos[keep_idx]
    M = p.shape[0]
    npad = ((M + 127) // 128) * 128
    pa = jnp.pad(p, ((0, npad - M), (0, 0)))
    prt = pa.T
    grid = (npad // 128,)
    nbr_t, attr_t = pl.pallas_call(
        functools.partial(_knn_kernel, M, npad),
        grid=grid,
        in_specs=[
            pl.BlockSpec((npad, 3), lambda b: (0, 0)),
            pl.BlockSpec((3, BR), lambda b: (0, b)),
        ],
        out_specs=[
            pl.BlockSpec((K, BR), lambda b: (0, b)),
            pl.BlockSpec((K, NUM_BINS, BR), lambda b: (0, 0, b)),
        ],
        out_shape=[
            jax.ShapeDtypeStruct((K, npad), jnp.int32),
            jax.ShapeDtypeStruct((K, NUM_BINS, npad), jnp.float32),
        ],
        compiler_params=pltpu.CompilerParams(
            dimension_semantics=("parallel",)),
    )(pa, prt)

    nbr = nbr_t[:, :M].T
    src = nbr.reshape(-1)
    dst = jnp.repeat(jnp.arange(M, dtype=jnp.int32), K)
    edge_index = jnp.stack([jnp.concatenate([src, dst]),
                            jnp.concatenate([dst, src])])
    A = attr_t[:, :, :M].transpose(2, 0, 1).reshape(-1, NUM_BINS)
    edge_attr = jnp.concatenate([A, A], axis=0)
    return p, edge_index, edge_attr
